# initial kernel scaffold (unmeasured)
import jax
import jax.numpy as jnp
from jax import lax
from jax.experimental import pallas as pl
from jax.experimental.pallas import tpu as pltpu

N_DEV = 16
M, N = 4096, 8192
CH = M // N_DEV


def kernel(x, w_mat, scale_x, scale_w):
    partial = lax.dot(
        x.astype(jnp.bfloat16),
        w_mat.astype(jnp.bfloat16),
        preferred_element_type=jnp.float32,
    )
    scale = (scale_x * scale_w).reshape(1)

    def body(s_ref, p_ref, o_ref, acc, rcv, loc, send_sems, recv_sems,
             loc_sem, out_sem, credit):
        d = lax.axis_index("i")
        left = lax.rem(d - 1 + N_DEV, N_DEV)
        right = lax.rem(d + 1, N_DEV)

        barrier = pltpu.get_barrier_semaphore()
        for nbr in (left, right):
            pl.semaphore_signal(barrier, inc=1, device_id=(nbr,),
                                device_id_type=pl.DeviceIdType.MESH)
        pl.semaphore_wait(barrier, 2)

        s = s_ref[0]

        cp = pltpu.make_async_copy(
            p_ref.at[pl.ds(d * CH, CH), :], loc, loc_sem)
        cp.start()
        cp.wait()
        acc[0] = loc[...] * s

        for h in range(2 * N_DEV - 2):
            cur, nxt = h % 2, (h + 1) % 2
            if h >= 2:
                pl.semaphore_wait(credit, 1)
            rdma = pltpu.make_async_remote_copy(
                src_ref=acc.at[cur],
                dst_ref=rcv.at[nxt],
                send_sem=send_sems.at[cur],
                recv_sem=recv_sems.at[nxt],
                device_id=(right,),
                device_id_type=pl.DeviceIdType.MESH,
            )
            rdma.start()
            if h < N_DEV - 1:
                c = lax.rem(d - h - 1 + N_DEV, N_DEV)
                cp = pltpu.make_async_copy(
                    p_ref.at[pl.ds(c * CH, CH), :], loc, loc_sem)
                cp.start()
            rdma.wait()
            if h < N_DEV - 2:
                cp.wait()
                acc[nxt] = rcv[nxt] + loc[...] * s
            elif h == N_DEV - 2:
                cp.wait()
                acc[nxt] = jnp.maximum(rcv[nxt] + loc[...] * s, 0.0)
                o = lax.rem(d + 1, N_DEV)
                st = pltpu.make_async_copy(
                    acc.at[nxt], o_ref.at[pl.ds(o * CH, CH), :], out_sem)
                st.start()
                st.wait()
            else:
                acc[nxt] = rcv[nxt]
                c = lax.rem(d - h + N_DEV - 1 + N_DEV, N_DEV)
                st = pltpu.make_async_copy(
                    acc.at[nxt], o_ref.at[pl.ds(c * CH, CH), :], out_sem)
                st.start()
                st.wait()
            if h <= 2 * N_DEV - 4:
                pl.semaphore_signal(credit, inc=1, device_id=(left,),
                                    device_id_type=pl.DeviceIdType.MESH)

    out = pl.pallas_call(
        body,
        out_shape=jax.ShapeDtypeStruct((M, N), jnp.float32),
        in_specs=[
            pl.BlockSpec(memory_space=pltpu.SMEM),
            pl.BlockSpec(memory_space=pltpu.ANY),
        ],
        out_specs=pl.BlockSpec(memory_space=pltpu.ANY),
        scratch_shapes=[
            pltpu.VMEM((2, CH, N), jnp.float32),
            pltpu.VMEM((2, CH, N), jnp.float32),
            pltpu.VMEM((CH, N), jnp.float32),
            pltpu.SemaphoreType.DMA((2,)),
            pltpu.SemaphoreType.DMA((2,)),
            pltpu.SemaphoreType.DMA,
            pltpu.SemaphoreType.DMA,
            pltpu.SemaphoreType.REGULAR,
        ],
        compiler_params=pltpu.CompilerParams(collective_id=0),
    )(scale, partial)
    return out


# baseline (device time: 2964733 ns/iter reference)
import jax
import jax.numpy as jnp
from jax import lax
from jax.experimental import pallas as pl
from jax.experimental.pallas import tpu as pltpu

N_DEV = 16
M, N = 4096, 8192
CH = M // N_DEV


def kernel(x, w_mat, scale_x, scale_w):
    partial = lax.dot(
        x.astype(jnp.bfloat16),
        w_mat.astype(jnp.bfloat16),
        preferred_element_type=jnp.float32,
    )
    scale = (scale_x * scale_w).reshape(1)

    def body(s_ref, p_ref, o_ref, acc, rcv, loc, send_sems, recv_sems,
             loc_sem, out_sem, credit):
        d = lax.axis_index("i")
        left = lax.rem(d - 1 + N_DEV, N_DEV)
        right = lax.rem(d + 1, N_DEV)

        barrier = pltpu.get_barrier_semaphore()
        for nbr in (left, right):
            pl.semaphore_signal(barrier, inc=1, device_id=(nbr,),
                                device_id_type=pl.DeviceIdType.MESH)
        pl.semaphore_wait(barrier, 2)

        s = s_ref[0]

        cp = pltpu.make_async_copy(
            p_ref.at[pl.ds(d * CH, CH), :], loc, loc_sem)
        cp.start()
        cp.wait()
        acc[0] = loc[...] * s

        for h in range(2 * N_DEV - 2):
            cur, nxt = h % 2, (h + 1) % 2
            if h >= 2:
                pl.semaphore_wait(credit, 1)
            rdma = pltpu.make_async_remote_copy(
                src_ref=acc.at[cur],
                dst_ref=rcv.at[nxt],
                send_sem=send_sems.at[cur],
                recv_sem=recv_sems.at[nxt],
                device_id=(right,),
                device_id_type=pl.DeviceIdType.MESH,
            )
            rdma.start()
            if h < N_DEV - 1:
                c = lax.rem(d - h - 1 + N_DEV, N_DEV)
                cp = pltpu.make_async_copy(
                    p_ref.at[pl.ds(c * CH, CH), :], loc, loc_sem)
                cp.start()
            rdma.wait()
            if h < N_DEV - 2:
                cp.wait()
                acc[nxt] = rcv[nxt] + loc[...] * s
            elif h == N_DEV - 2:
                cp.wait()
                acc[nxt] = jnp.maximum(rcv[nxt] + loc[...] * s, 0.0)
                o = lax.rem(d + 1, N_DEV)
                st = pltpu.make_async_copy(
                    acc.at[nxt], o_ref.at[pl.ds(o * CH, CH), :], out_sem)
                st.start()
                st.wait()
            else:
                acc[nxt] = rcv[nxt]
                c = lax.rem(d - h + N_DEV - 1 + N_DEV, N_DEV)
                st = pltpu.make_async_copy(
                    acc.at[nxt], o_ref.at[pl.ds(c * CH, CH), :], out_sem)
                st.start()
                st.wait()
            if h <= 2 * N_DEV - 5:
                pl.semaphore_signal(credit, inc=1, device_id=(left,),
                                    device_id_type=pl.DeviceIdType.MESH)

    out = pl.pallas_call(
        body,
        out_shape=jax.ShapeDtypeStruct((M, N), jnp.float32),
        in_specs=[
            pl.BlockSpec(memory_space=pltpu.SMEM),
            pl.BlockSpec(memory_space=pl.ANY),
        ],
        out_specs=pl.BlockSpec(memory_space=pl.ANY),
        scratch_shapes=[
            pltpu.VMEM((2, CH, N), jnp.float32),
            pltpu.VMEM((2, CH, N), jnp.float32),
            pltpu.VMEM((CH, N), jnp.float32),
            pltpu.SemaphoreType.DMA((2,)),
            pltpu.SemaphoreType.DMA((2,)),
            pltpu.SemaphoreType.DMA,
            pltpu.SemaphoreType.DMA,
            pltpu.SemaphoreType.REGULAR,
        ],
        compiler_params=pltpu.CompilerParams(
            collective_id=0, vmem_limit_bytes=60 * 1024 * 1024),
    )(scale, partial)
    return out


# device time: 1652421 ns/iter; 1.7942x vs baseline; 1.7942x over previous
import jax
import jax.numpy as jnp
from jax import lax
from jax.experimental import pallas as pl
from jax.experimental.pallas import tpu as pltpu

N_DEV = 16
M, N = 4096, 8192
CH = M // N_DEV
HN = N // 2
N_HOP = 2 * N_DEV - 2


def kernel(x, w_mat, scale_x, scale_w):
    partial = lax.dot(
        x.astype(jnp.bfloat16),
        w_mat.astype(jnp.bfloat16),
        preferred_element_type=jnp.float32,
    )
    scale = (scale_x * scale_w).reshape(1)

    def body(s_ref, p_ref, o_ref,
             accA, accB, rcvA, rcvB, locA, locB,
             sendA, recvA, sendB, recvB,
             lsemA, lsemB, osemA, osemB, credA, credB):
        d = lax.axis_index("i")
        left = lax.rem(d - 1 + N_DEV, N_DEV)
        right = lax.rem(d + 1, N_DEV)

        barrier = pltpu.get_barrier_semaphore()
        for nbr in (left, right):
            pl.semaphore_signal(barrier, inc=1, device_id=(nbr,),
                                device_id_type=pl.DeviceIdType.MESH)
        pl.semaphore_wait(barrier, 2)

        s = s_ref[0]

        def load_half(chunk, ring):
            loc, sem, c0 = (locA, lsemA, 0) if ring == 0 else (locB, lsemB, HN)
            cp = pltpu.make_async_copy(
                p_ref.at[pl.ds(chunk * CH, CH), pl.ds(c0, HN)], loc, sem)
            cp.start()
            return cp

        def store_half(src, chunk, ring):
            sem, c0 = (osemA, 0) if ring == 0 else (osemB, HN)
            st = pltpu.make_async_copy(
                src, o_ref.at[pl.ds(chunk * CH, CH), pl.ds(c0, HN)], sem)
            st.start()
            return st

        cpA = load_half(d, 0)
        cpB = load_half(d, 1)
        cpA.wait()
        accA[0] = locA[...] * s
        cpB.wait()
        accB[0] = locB[...] * s

        for h in range(N_HOP):
            cur, nxt = h % 2, (h + 1) % 2
            if h >= 2:
                pl.semaphore_wait(credA, 1)
                pl.semaphore_wait(credB, 1)
            rdmaA = pltpu.make_async_remote_copy(
                src_ref=accA.at[cur], dst_ref=rcvA.at[nxt],
                send_sem=sendA.at[cur], recv_sem=recvA.at[nxt],
                device_id=(right,), device_id_type=pl.DeviceIdType.MESH)
            rdmaB = pltpu.make_async_remote_copy(
                src_ref=accB.at[cur], dst_ref=rcvB.at[nxt],
                send_sem=sendB.at[cur], recv_sem=recvB.at[nxt],
                device_id=(left,), device_id_type=pl.DeviceIdType.MESH)
            rdmaA.start()
            rdmaB.start()
            if h < N_DEV - 1:
                cA = lax.rem(d - h - 1 + N_DEV, N_DEV)
                cB = lax.rem(d + h + 1, N_DEV)
                cpA = load_half(cA, 0)
                cpB = load_half(cB, 1)
            rdmaA.wait()
            rdmaB.wait()
            if h < N_DEV - 2:
                cpA.wait()
                accA[nxt] = rcvA[nxt] + locA[...] * s
                cpB.wait()
                accB[nxt] = rcvB[nxt] + locB[...] * s
            elif h == N_DEV - 2:
                cpA.wait()
                accA[nxt] = jnp.maximum(rcvA[nxt] + locA[...] * s, 0.0)
                stA = store_half(accA.at[nxt], lax.rem(d + 1, N_DEV), 0)
                cpB.wait()
                accB[nxt] = jnp.maximum(rcvB[nxt] + locB[...] * s, 0.0)
                stB = store_half(accB.at[nxt], lax.rem(d - 1 + N_DEV, N_DEV), 1)
                stA.wait()
                stB.wait()
            else:
                accA[nxt] = rcvA[nxt]
                accB[nxt] = rcvB[nxt]
                cA = lax.rem(d - h + N_DEV - 1 + N_DEV, N_DEV)
                cB = lax.rem(d + h - N_DEV + 1 + N_DEV, N_DEV)
                stA = store_half(accA.at[nxt], cA, 0)
                stB = store_half(accB.at[nxt], cB, 1)
                stA.wait()
                stB.wait()
            if h <= N_HOP - 3:
                pl.semaphore_signal(credA, inc=1, device_id=(left,),
                                    device_id_type=pl.DeviceIdType.MESH)
                pl.semaphore_signal(credB, inc=1, device_id=(right,),
                                    device_id_type=pl.DeviceIdType.MESH)

    out = pl.pallas_call(
        body,
        out_shape=jax.ShapeDtypeStruct((M, N), jnp.float32),
        in_specs=[
            pl.BlockSpec(memory_space=pltpu.SMEM),
            pl.BlockSpec(memory_space=pl.ANY),
        ],
        out_specs=pl.BlockSpec(memory_space=pl.ANY),
        scratch_shapes=[
            pltpu.VMEM((2, CH, HN), jnp.float32),
            pltpu.VMEM((2, CH, HN), jnp.float32),
            pltpu.VMEM((2, CH, HN), jnp.float32),
            pltpu.VMEM((2, CH, HN), jnp.float32),
            pltpu.VMEM((CH, HN), jnp.float32),
            pltpu.VMEM((CH, HN), jnp.float32),
            pltpu.SemaphoreType.DMA((2,)),
            pltpu.SemaphoreType.DMA((2,)),
            pltpu.SemaphoreType.DMA((2,)),
            pltpu.SemaphoreType.DMA((2,)),
            pltpu.SemaphoreType.DMA,
            pltpu.SemaphoreType.DMA,
            pltpu.SemaphoreType.DMA,
            pltpu.SemaphoreType.DMA,
            pltpu.SemaphoreType.REGULAR,
            pltpu.SemaphoreType.REGULAR,
        ],
        compiler_params=pltpu.CompilerParams(
            collective_id=0, vmem_limit_bytes=60 * 1024 * 1024),
    )(scale, partial)
    return out


# device time: 956162 ns/iter; 3.1007x vs baseline; 1.7282x over previous
import jax
import jax.numpy as jnp
from jax import lax
from jax.experimental import pallas as pl
from jax.experimental.pallas import tpu as pltpu

N_DEV = 16
M, N = 4096, 8192
CH = M // N_DEV
HN = N // 2
N_HOP = 2 * N_DEV - 2


def kernel(x, w_mat, scale_x, scale_w):
    partial = lax.dot(
        x.astype(jnp.bfloat16),
        w_mat.astype(jnp.bfloat16),
        preferred_element_type=jnp.float32,
    )
    partial = (partial * (scale_x * scale_w)[0]).astype(jnp.bfloat16)

    def body(p_ref, o_ref,
             accA, accB, rcvA, rcvB, locA, locB, stfA, stfB,
             sendA, recvA, sendB, recvB,
             lsemA, lsemB, osemA, osemB, credA, credB):
        d = lax.axis_index("i")
        left = lax.rem(d - 1 + N_DEV, N_DEV)
        right = lax.rem(d + 1, N_DEV)

        barrier = pltpu.get_barrier_semaphore()
        for nbr in (left, right):
            pl.semaphore_signal(barrier, inc=1, device_id=(nbr,),
                                device_id_type=pl.DeviceIdType.MESH)
        pl.semaphore_wait(barrier, 2)

        def load_half(chunk, ring):
            loc, sem, c0 = (locA, lsemA, 0) if ring == 0 else (locB, lsemB, HN)
            cp = pltpu.make_async_copy(
                p_ref.at[pl.ds(chunk * CH, CH), pl.ds(c0, HN)], loc, sem)
            cp.start()
            return cp

        def store_half(src, chunk, ring):
            sem, c0 = (osemA, 0) if ring == 0 else (osemB, HN)
            st = pltpu.make_async_copy(
                src, o_ref.at[pl.ds(chunk * CH, CH), pl.ds(c0, HN)], sem)
            st.start()
            return st

        cpA = load_half(d, 0)
        cpB = load_half(d, 1)
        cpA.wait()
        accA[0] = locA[...]
        cpB.wait()
        accB[0] = locB[...]

        for h in range(N_HOP):
            cur, nxt = h % 2, (h + 1) % 2
            if h >= 2:
                pl.semaphore_wait(credA, 1)
                pl.semaphore_wait(credB, 1)
            rdmaA = pltpu.make_async_remote_copy(
                src_ref=accA.at[cur], dst_ref=rcvA.at[nxt],
                send_sem=sendA.at[cur], recv_sem=recvA.at[nxt],
                device_id=(right,), device_id_type=pl.DeviceIdType.MESH)
            rdmaB = pltpu.make_async_remote_copy(
                src_ref=accB.at[cur], dst_ref=rcvB.at[nxt],
                send_sem=sendB.at[cur], recv_sem=recvB.at[nxt],
                device_id=(left,), device_id_type=pl.DeviceIdType.MESH)
            rdmaA.start()
            rdmaB.start()
            if h < N_DEV - 1:
                cA = lax.rem(d - h - 1 + N_DEV, N_DEV)
                cB = lax.rem(d + h + 1, N_DEV)
                cpA = load_half(cA, 0)
                cpB = load_half(cB, 1)
            rdmaA.wait()
            rdmaB.wait()
            if h < N_DEV - 2:
                cpA.wait()
                accA[nxt] = rcvA[nxt] + locA[...]
                cpB.wait()
                accB[nxt] = rcvB[nxt] + locB[...]
            elif h == N_DEV - 2:
                cpA.wait()
                accA[nxt] = jnp.maximum(rcvA[nxt] + locA[...], 0)
                stfA[...] = accA[nxt].astype(jnp.float32)
                stA = store_half(stfA, lax.rem(d + 1, N_DEV), 0)
                cpB.wait()
                accB[nxt] = jnp.maximum(rcvB[nxt] + locB[...], 0)
                stfB[...] = accB[nxt].astype(jnp.float32)
                stB = store_half(stfB, lax.rem(d - 1 + N_DEV, N_DEV), 1)
                stA.wait()
                stB.wait()
            else:
                accA[nxt] = rcvA[nxt]
                accB[nxt] = rcvB[nxt]
                stfA[...] = rcvA[nxt].astype(jnp.float32)
                stfB[...] = rcvB[nxt].astype(jnp.float32)
                cA = lax.rem(d - h + N_DEV - 1 + N_DEV, N_DEV)
                cB = lax.rem(d + h + 1, N_DEV)
                stA = store_half(stfA, cA, 0)
                stB = store_half(stfB, cB, 1)
                stA.wait()
                stB.wait()
            if h <= N_HOP - 3:
                pl.semaphore_signal(credA, inc=1, device_id=(left,),
                                    device_id_type=pl.DeviceIdType.MESH)
                pl.semaphore_signal(credB, inc=1, device_id=(right,),
                                    device_id_type=pl.DeviceIdType.MESH)

    out = pl.pallas_call(
        body,
        out_shape=jax.ShapeDtypeStruct((M, N), jnp.float32),
        in_specs=[pl.BlockSpec(memory_space=pl.ANY)],
        out_specs=pl.BlockSpec(memory_space=pl.ANY),
        scratch_shapes=[
            pltpu.VMEM((2, CH, HN), jnp.bfloat16),
            pltpu.VMEM((2, CH, HN), jnp.bfloat16),
            pltpu.VMEM((2, CH, HN), jnp.bfloat16),
            pltpu.VMEM((2, CH, HN), jnp.bfloat16),
            pltpu.VMEM((CH, HN), jnp.bfloat16),
            pltpu.VMEM((CH, HN), jnp.bfloat16),
            pltpu.VMEM((CH, HN), jnp.float32),
            pltpu.VMEM((CH, HN), jnp.float32),
            pltpu.SemaphoreType.DMA((2,)),
            pltpu.SemaphoreType.DMA((2,)),
            pltpu.SemaphoreType.DMA((2,)),
            pltpu.SemaphoreType.DMA((2,)),
            pltpu.SemaphoreType.DMA,
            pltpu.SemaphoreType.DMA,
            pltpu.SemaphoreType.DMA,
            pltpu.SemaphoreType.DMA,
            pltpu.SemaphoreType.REGULAR,
            pltpu.SemaphoreType.REGULAR,
        ],
        compiler_params=pltpu.CompilerParams(
            collective_id=0, vmem_limit_bytes=60 * 1024 * 1024),
    )(partial)
    return out


# device time: 888000 ns/iter; 3.3387x vs baseline; 1.0768x over previous
import jax
import jax.numpy as jnp
from jax import lax
from jax.experimental import pallas as pl
from jax.experimental.pallas import tpu as pltpu

N_DEV = 16
M, N = 4096, 8192
CH = M // N_DEV
HN = N // 2
N_HOP = 2 * N_DEV - 2


def kernel(x, w_mat, scale_x, scale_w):
    scale = (scale_x * scale_w).reshape(1)

    def body(s_ref, x_ref, w_ref, o_ref,
             w_bf, accA, accB, rcvA, rcvB, stfA, stfB,
             sendA, recvA, sendB, recvB,
             osemA, osemB, credA, credB):
        d = lax.axis_index("i")
        left = lax.rem(d - 1 + N_DEV, N_DEV)
        right = lax.rem(d + 1, N_DEV)

        barrier = pltpu.get_barrier_semaphore()
        for nbr in (left, right):
            pl.semaphore_signal(barrier, inc=1, device_id=(nbr,),
                                device_id_type=pl.DeviceIdType.MESH)
        pl.semaphore_wait(barrier, 2)

        s = s_ref[0]
        w_bf[...] = w_ref[...].astype(jnp.bfloat16)

        def chunk_gemm(chunk, ring):
            xc = x_ref[pl.ds(chunk * CH, CH), :].astype(jnp.bfloat16)
            wc = w_bf[:, pl.ds(0 if ring == 0 else HN, HN)]
            p = lax.dot(xc, wc, preferred_element_type=jnp.float32)
            return (p * s).astype(jnp.bfloat16)

        def store_half(src, chunk, ring):
            sem, c0 = (osemA, 0) if ring == 0 else (osemB, HN)
            st = pltpu.make_async_copy(
                src, o_ref.at[pl.ds(chunk * CH, CH), pl.ds(c0, HN)], sem)
            st.start()
            return st

        accA[0] = chunk_gemm(d, 0)
        accB[0] = chunk_gemm(d, 1)

        st_pend = {0: [None, None], 1: [None, None]}

        for h in range(N_HOP):
            cur, nxt = h % 2, (h + 1) % 2
            if h >= 2:
                pl.semaphore_wait(credA, 1)
                pl.semaphore_wait(credB, 1)
            rdmaA = pltpu.make_async_remote_copy(
                src_ref=accA.at[cur], dst_ref=rcvA.at[nxt],
                send_sem=sendA.at[cur], recv_sem=recvA.at[nxt],
                device_id=(right,), device_id_type=pl.DeviceIdType.MESH)
            rdmaB = pltpu.make_async_remote_copy(
                src_ref=accB.at[cur], dst_ref=rcvB.at[nxt],
                send_sem=sendB.at[cur], recv_sem=recvB.at[nxt],
                device_id=(left,), device_id_type=pl.DeviceIdType.MESH)
            rdmaA.start()
            rdmaB.start()
            if h < N_DEV - 1:
                pA = chunk_gemm(lax.rem(d - h - 1 + N_DEV, N_DEV), 0)
                pB = chunk_gemm(lax.rem(d + h + 1, N_DEV), 1)
            for ring in (0, 1):
                if st_pend[ring][nxt] is not None:
                    st_pend[ring][nxt].wait()
                    st_pend[ring][nxt] = None
            rdmaA.wait()
            rdmaB.wait()
            if h < N_DEV - 2:
                accA[nxt] = rcvA[nxt] + pA
                accB[nxt] = rcvB[nxt] + pB
            elif h == N_DEV - 2:
                accA[nxt] = jnp.maximum(rcvA[nxt] + pA, 0)
                accB[nxt] = jnp.maximum(rcvB[nxt] + pB, 0)
                stfA[nxt] = accA[nxt].astype(jnp.float32)
                stfB[nxt] = accB[nxt].astype(jnp.float32)
            else:
                accA[nxt] = rcvA[nxt]
                accB[nxt] = rcvB[nxt]
                stfA[nxt] = rcvA[nxt].astype(jnp.float32)
                stfB[nxt] = rcvB[nxt].astype(jnp.float32)
            if h <= N_HOP - 3:
                pl.semaphore_signal(credA, inc=1, device_id=(left,),
                                    device_id_type=pl.DeviceIdType.MESH)
                pl.semaphore_signal(credB, inc=1, device_id=(right,),
                                    device_id_type=pl.DeviceIdType.MESH)
            if h == N_DEV - 2:
                st_pend[0][nxt] = store_half(
                    stfA.at[nxt], lax.rem(d + 1, N_DEV), 0)
                st_pend[1][nxt] = store_half(
                    stfB.at[nxt], lax.rem(d - 1 + N_DEV, N_DEV), 1)
            elif h > N_DEV - 2:
                st_pend[0][nxt] = store_half(
                    stfA.at[nxt], lax.rem(d - h + N_DEV - 1 + N_DEV, N_DEV), 0)
                st_pend[1][nxt] = store_half(
                    stfB.at[nxt], lax.rem(d + h + 1, N_DEV), 1)

        for ring in (0, 1):
            for slot in (0, 1):
                if st_pend[ring][slot] is not None:
                    st_pend[ring][slot].wait()

    out = pl.pallas_call(
        body,
        out_shape=jax.ShapeDtypeStruct((M, N), jnp.float32),
        in_specs=[
            pl.BlockSpec(memory_space=pltpu.SMEM),
            pl.BlockSpec(memory_space=pltpu.VMEM),
            pl.BlockSpec(memory_space=pltpu.VMEM),
        ],
        out_specs=pl.BlockSpec(memory_space=pl.ANY),
        scratch_shapes=[
            pltpu.VMEM((M // N_DEV, N), jnp.bfloat16),
            pltpu.VMEM((2, CH, HN), jnp.bfloat16),
            pltpu.VMEM((2, CH, HN), jnp.bfloat16),
            pltpu.VMEM((2, CH, HN), jnp.bfloat16),
            pltpu.VMEM((2, CH, HN), jnp.bfloat16),
            pltpu.VMEM((2, CH, HN), jnp.float32),
            pltpu.VMEM((2, CH, HN), jnp.float32),
            pltpu.SemaphoreType.DMA((2,)),
            pltpu.SemaphoreType.DMA((2,)),
            pltpu.SemaphoreType.DMA((2,)),
            pltpu.SemaphoreType.DMA((2,)),
            pltpu.SemaphoreType.DMA,
            pltpu.SemaphoreType.DMA,
            pltpu.SemaphoreType.REGULAR,
            pltpu.SemaphoreType.REGULAR,
        ],
        compiler_params=pltpu.CompilerParams(
            collective_id=0, vmem_limit_bytes=60 * 1024 * 1024),
    )(scale, x, w_mat)
    return out


# device time: 838269 ns/iter; 3.5367x vs baseline; 1.0593x over previous
import jax
import jax.numpy as jnp
from jax import lax
from jax.experimental import pallas as pl
from jax.experimental.pallas import tpu as pltpu

N_DEV = 16
M, N = 4096, 8192
CH = M // N_DEV
HN = N // 2
N_HOP = 2 * N_DEV - 2


def kernel(x, w_mat, scale_x, scale_w):
    scale = (scale_x * scale_w).reshape(1)

    def body(s_ref, x_ref, w_ref, o_ref,
             w_bf, accA, accB, rcvA, rcvB, stfA, stfB,
             sendA, recvA, sendB, recvB,
             osemA, osemB, credA, credB):
        d = lax.axis_index("i")
        left = lax.rem(d - 1 + N_DEV, N_DEV)
        right = lax.rem(d + 1, N_DEV)

        barrier = pltpu.get_barrier_semaphore()
        for nbr in (left, right):
            pl.semaphore_signal(barrier, inc=1, device_id=(nbr,),
                                device_id_type=pl.DeviceIdType.MESH)
        pl.semaphore_wait(barrier, 2)

        s = s_ref[0]
        w_bf[...] = w_ref[...].astype(jnp.bfloat16)

        def chunk_gemm(chunk, ring):
            xc = x_ref[pl.ds(chunk * CH, CH), :].astype(jnp.bfloat16)
            wc = w_bf[:, pl.ds(0 if ring == 0 else HN, HN)]
            p = lax.dot(xc, wc, preferred_element_type=jnp.float32)
            return (p * s).astype(jnp.bfloat16)

        def store_half(src, chunk, ring):
            sem, c0 = (osemA, 0) if ring == 0 else (osemB, HN)
            st = pltpu.make_async_copy(
                src, o_ref.at[pl.ds(chunk * CH, CH), pl.ds(c0, HN)], sem)
            st.start()
            return st

        def mk_rdma(h, ring):
            cur, nxt = h % 2, (h + 1) % 2
            acc, rcv, snd, rec, nbr = (
                (accA, rcvA, sendA, recvA, right) if ring == 0
                else (accB, rcvB, sendB, recvB, left))
            return pltpu.make_async_remote_copy(
                src_ref=acc.at[cur], dst_ref=rcv.at[nxt],
                send_sem=snd.at[cur], recv_sem=rec.at[nxt],
                device_id=(nbr,), device_id_type=pl.DeviceIdType.MESH)

        accA[0] = chunk_gemm(d, 0)
        accB[0] = chunk_gemm(d, 1)
        rdA = mk_rdma(0, 0)
        rdB = mk_rdma(0, 1)
        rdA.start()
        rdB.start()

        st_pend = {0: [None, None], 1: [None, None]}

        def consume(h, ring):
            cur, nxt = h % 2, (h + 1) % 2
            acc, rcv, stf, p = (
                (accA, rcvA, stfA, None) if ring == 0
                else (accB, rcvB, stfB, None))
            if h < N_DEV - 2:
                acc[nxt] = rcv[nxt] + (pA if ring == 0 else pB)
            else:
                if st_pend[ring][nxt] is not None:
                    st_pend[ring][nxt].wait()
                    st_pend[ring][nxt] = None
                if h == N_DEV - 2:
                    acc[nxt] = jnp.maximum(
                        rcv[nxt] + (pA if ring == 0 else pB), 0)
                    stf[nxt] = acc[nxt].astype(jnp.float32)
                    c = lax.rem(d + 1, N_DEV) if ring == 0 \
                        else lax.rem(d - 1 + N_DEV, N_DEV)
                else:
                    acc[nxt] = rcv[nxt]
                    stf[nxt] = rcv[nxt].astype(jnp.float32)
                    c = lax.rem(d - h + N_DEV - 1 + N_DEV, N_DEV) \
                        if ring == 0 else lax.rem(d + h + 1, N_DEV)
                st_pend[ring][nxt] = store_half(stf.at[nxt], c, ring)

        for h in range(N_HOP):
            if h < N_DEV - 1:
                pA = chunk_gemm(lax.rem(d - h - 1 + N_DEV, N_DEV), 0)
                pB = chunk_gemm(lax.rem(d + h + 1, N_DEV), 1)
            rdA.wait()
            consume(h, 0)
            if h <= N_HOP - 3:
                pl.semaphore_signal(credA, inc=1, device_id=(left,),
                                    device_id_type=pl.DeviceIdType.MESH)
            if h + 1 < N_HOP:
                if h + 1 >= 2:
                    pl.semaphore_wait(credA, 1)
                rdA = mk_rdma(h + 1, 0)
                rdA.start()
            rdB.wait()
            consume(h, 1)
            if h <= N_HOP - 3:
                pl.semaphore_signal(credB, inc=1, device_id=(right,),
                                    device_id_type=pl.DeviceIdType.MESH)
            if h + 1 < N_HOP:
                if h + 1 >= 2:
                    pl.semaphore_wait(credB, 1)
                rdB = mk_rdma(h + 1, 1)
                rdB.start()

        for ring in (0, 1):
            for slot in (0, 1):
                if st_pend[ring][slot] is not None:
                    st_pend[ring][slot].wait()

    out = pl.pallas_call(
        body,
        out_shape=jax.ShapeDtypeStruct((M, N), jnp.float32),
        in_specs=[
            pl.BlockSpec(memory_space=pltpu.SMEM),
            pl.BlockSpec(memory_space=pltpu.VMEM),
            pl.BlockSpec(memory_space=pltpu.VMEM),
        ],
        out_specs=pl.BlockSpec(memory_space=pl.ANY),
        scratch_shapes=[
            pltpu.VMEM((M // N_DEV, N), jnp.bfloat16),
            pltpu.VMEM((2, CH, HN), jnp.bfloat16),
            pltpu.VMEM((2, CH, HN), jnp.bfloat16),
            pltpu.VMEM((2, CH, HN), jnp.bfloat16),
            pltpu.VMEM((2, CH, HN), jnp.bfloat16),
            pltpu.VMEM((2, CH, HN), jnp.float32),
            pltpu.VMEM((2, CH, HN), jnp.float32),
            pltpu.SemaphoreType.DMA((2,)),
            pltpu.SemaphoreType.DMA((2,)),
            pltpu.SemaphoreType.DMA((2,)),
            pltpu.SemaphoreType.DMA((2,)),
            pltpu.SemaphoreType.DMA,
            pltpu.SemaphoreType.DMA,
            pltpu.SemaphoreType.REGULAR,
            pltpu.SemaphoreType.REGULAR,
        ],
        compiler_params=pltpu.CompilerParams(
            collective_id=0, vmem_limit_bytes=60 * 1024 * 1024),
    )(scale, x, w_mat)
    return out


# device time: 832925 ns/iter; 3.5594x vs baseline; 1.0064x over previous
import jax
import jax.numpy as jnp
from jax import lax
from jax.experimental import pallas as pl
from jax.experimental.pallas import tpu as pltpu

N_DEV = 16
M, N = 4096, 8192
CH = M // N_DEV
HN = N // 2
HQ = HN // 2
N_HOP = 2 * N_DEV - 2


def kernel(x, w_mat, scale_x, scale_w):
    scale = (scale_x * scale_w).reshape(1)

    def body(s_ref, x_ref, w_ref, o_ref,
             w_bf, accA, accB, rcvA, rcvB, stfA, stfB,
             sendA, recvA, sendB, recvB,
             osemA, osemB, credA, credB):
        d = lax.axis_index("i")
        left = lax.rem(d - 1 + N_DEV, N_DEV)
        right = lax.rem(d + 1, N_DEV)

        barrier = pltpu.get_barrier_semaphore()
        for nbr in (left, right):
            pl.semaphore_signal(barrier, inc=1, device_id=(nbr,),
                                device_id_type=pl.DeviceIdType.MESH)
        pl.semaphore_wait(barrier, 2)

        s = s_ref[0]
        w_bf[...] = w_ref[...].astype(jnp.bfloat16)

        rings = {
            0: dict(acc=accA, rcv=rcvA, stf=stfA, snd=sendA, rec=recvA,
                    osem=osemA, cred=credA, to=right, frm=left, c0=0),
            1: dict(acc=accB, rcv=rcvB, stf=stfB, snd=sendB, rec=recvB,
                    osem=osemB, cred=credB, to=left, frm=right, c0=HN),
        }

        def chunk_gemm(chunk, ring):
            xc = x_ref[pl.ds(chunk * CH, CH), :].astype(jnp.bfloat16)
            wc = w_bf[:, pl.ds(rings[ring]["c0"], HN)]
            p = lax.dot(xc, wc, preferred_element_type=jnp.float32)
            return (p * s).astype(jnp.bfloat16)

        def mk_rdma(h, ring, sub):
            r = rings[ring]
            cur, nxt = h % 2, (h + 1) % 2
            return pltpu.make_async_remote_copy(
                src_ref=r["acc"].at[cur, sub],
                dst_ref=r["rcv"].at[nxt, sub],
                send_sem=r["snd"].at[cur, sub],
                recv_sem=r["rec"].at[nxt, sub],
                device_id=(r["to"],), device_id_type=pl.DeviceIdType.MESH)

        def launch(h, ring):
            a = mk_rdma(h, ring, 0)
            b = mk_rdma(h, ring, 1)
            a.start()
            b.start()
            return a, b

        def store_chunk(h, ring):
            r = rings[ring]
            nxt = (h + 1) % 2
            if h == N_DEV - 2:
                c = lax.rem(d + 1, N_DEV) if ring == 0 \
                    else lax.rem(d - 1 + N_DEV, N_DEV)
            else:
                c = lax.rem(d - h + N_DEV - 1 + N_DEV, N_DEV) \
                    if ring == 0 else lax.rem(d + h + 1, N_DEV)
            sts = []
            for sub in (0, 1):
                st = pltpu.make_async_copy(
                    r["stf"].at[nxt, sub],
                    o_ref.at[pl.ds(c * CH, CH),
                             pl.ds(r["c0"] + sub * HQ, HQ)],
                    r["osem"])
                st.start()
                sts.append(st)
            return sts

        def consume_sub(h, ring, sub, p):
            r = rings[ring]
            nxt = (h + 1) % 2
            if h < N_DEV - 2:
                r["acc"][nxt, sub] = (
                    r["rcv"][nxt, sub] + p[:, sub * HQ:(sub + 1) * HQ])
            elif h == N_DEV - 2:
                v = jnp.maximum(
                    r["rcv"][nxt, sub] + p[:, sub * HQ:(sub + 1) * HQ], 0)
                r["acc"][nxt, sub] = v
                r["stf"][nxt, sub] = v.astype(jnp.float32)
            else:
                v = r["rcv"][nxt, sub]
                r["acc"][nxt, sub] = v
                r["stf"][nxt, sub] = v.astype(jnp.float32)

        for ring in (0, 1):
            p0 = chunk_gemm(d, ring)
            for sub in (0, 1):
                rings[ring]["acc"][0, sub] = p0[:, sub * HQ:(sub + 1) * HQ]
        rdA = launch(0, 0)
        rdB = launch(0, 1)

        st_pend = {0: [None, None], 1: [None, None]}

        for h in range(N_HOP):
            cur, nxt = h % 2, (h + 1) % 2
            pA = pB = None
            if h < N_DEV - 1:
                pA = chunk_gemm(lax.rem(d - h - 1 + N_DEV, N_DEV), 0)
                pB = chunk_gemm(lax.rem(d + h + 1, N_DEV), 1)
            for ring, rd in ((0, rdA), (1, rdB)):
                r = rings[ring]
                p = pA if ring == 0 else pB
                if h >= N_DEV - 2 and st_pend[ring][nxt] is not None:
                    for st in st_pend[ring][nxt]:
                        st.wait()
                    st_pend[ring][nxt] = None
                rd[0].wait()
                consume_sub(h, ring, 0, p)
                rd[1].wait()
                consume_sub(h, ring, 1, p)
                if h <= N_HOP - 3:
                    pl.semaphore_signal(
                        r["cred"], inc=1, device_id=(r["frm"],),
                        device_id_type=pl.DeviceIdType.MESH)
                if h + 1 < N_HOP:
                    if h + 1 >= 2:
                        pl.semaphore_wait(r["cred"], 1)
                    nrd = launch(h + 1, ring)
                    if ring == 0:
                        rdA = nrd
                    else:
                        rdB = nrd
                if h >= N_DEV - 2:
                    st_pend[ring][nxt] = store_chunk(h, ring)

        for ring in (0, 1):
            for slot in (0, 1):
                if st_pend[ring][slot] is not None:
                    for st in st_pend[ring][slot]:
                        st.wait()

    out = pl.pallas_call(
        body,
        out_shape=jax.ShapeDtypeStruct((M, N), jnp.float32),
        in_specs=[
            pl.BlockSpec(memory_space=pltpu.SMEM),
            pl.BlockSpec(memory_space=pltpu.VMEM),
            pl.BlockSpec(memory_space=pltpu.VMEM),
        ],
        out_specs=pl.BlockSpec(memory_space=pl.ANY),
        scratch_shapes=[
            pltpu.VMEM((M // N_DEV, N), jnp.bfloat16),
            pltpu.VMEM((2, 2, CH, HQ), jnp.bfloat16),
            pltpu.VMEM((2, 2, CH, HQ), jnp.bfloat16),
            pltpu.VMEM((2, 2, CH, HQ), jnp.bfloat16),
            pltpu.VMEM((2, 2, CH, HQ), jnp.bfloat16),
            pltpu.VMEM((2, 2, CH, HQ), jnp.float32),
            pltpu.VMEM((2, 2, CH, HQ), jnp.float32),
            pltpu.SemaphoreType.DMA((2, 2)),
            pltpu.SemaphoreType.DMA((2, 2)),
            pltpu.SemaphoreType.DMA((2, 2)),
            pltpu.SemaphoreType.DMA((2, 2)),
            pltpu.SemaphoreType.DMA,
            pltpu.SemaphoreType.DMA,
            pltpu.SemaphoreType.REGULAR,
            pltpu.SemaphoreType.REGULAR,
        ],
        compiler_params=pltpu.CompilerParams(
            collective_id=0, vmem_limit_bytes=60 * 1024 * 1024),
    )(scale, x, w_mat)
    return out
